# baseline (speedup 1.0000x reference)
import functools

import jax
import jax.numpy as jnp
from jax import lax
from jax.experimental import pallas as pl
from jax.experimental.pallas import tpu as pltpu
from jax.experimental.pallas import tpu_sc as plsc

VOCAB = 1000000
HIDDEN = 64
BATCH = 16384
HIST = 50

NC = 2
NS = 16
NW = NC * NS

TOTAL = BATCH * HIST
PER_W = TOTAL // NW
CHUNK = 128
NSTEPS = PER_W // CHUNK
NBUF = 4

PAD = 128
TBLK = 512


def _tc_table_transpose():
  def body(wt_ref, out_ref):
    out_ref[:, 0:HIDDEN] = wt_ref[...].T

  grid = (VOCAB + TBLK - 1) // TBLK
  return pl.pallas_call(
      body,
      grid=(grid,),
      in_specs=[pl.BlockSpec((HIDDEN, TBLK), lambda i: (0, i))],
      out_specs=pl.BlockSpec((TBLK, PAD), lambda i: (i, 0)),
      out_shape=jax.ShapeDtypeStruct((VOCAB, PAD), jnp.float32),
  )


def _sc_gather():
  mesh = plsc.VectorSubcoreMesh(
      core_axis_name="c", subcore_axis_name="s", num_cores=NC, num_subcores=NS
  )

  @functools.partial(
      pl.kernel,
      out_type=jax.ShapeDtypeStruct((TOTAL, HIDDEN), jnp.float32),
      mesh=mesh,
      scratch_types=[
          pltpu.VMEM((NSTEPS, CHUNK), jnp.int32),
          [pltpu.VMEM((CHUNK, PAD), jnp.float32) for _ in range(NBUF)],
          [pltpu.SemaphoreType.DMA for _ in range(NBUF)],
          [pltpu.SemaphoreType.DMA for _ in range(NBUF)],
      ],
      compiler_params=pltpu.CompilerParams(use_tc_tiling_on_sc=False),
  )
  def k(idx_hbm, table_hbm, out_hbm, idx_v, bufs, g_sems, w_sems):
    wid = lax.axis_index("s") * NC + lax.axis_index("c")
    base = wid * PER_W
    pltpu.sync_copy(idx_hbm.at[wid], idx_v)

    def gather(c, b):
      pltpu.async_copy(table_hbm.at[idx_v.at[c]], bufs[b], g_sems[b])

    def gather_wait(c, b):
      pltpu.make_async_copy(table_hbm.at[idx_v.at[c]], bufs[b], g_sems[b]).wait()

    def wb(c, b):
      pltpu.async_copy(
          bufs[b].at[:, pl.ds(0, HIDDEN)],
          out_hbm.at[pl.ds(base + c * CHUNK, CHUNK)],
          w_sems[b],
      )

    def wb_wait(c, b):
      pltpu.make_async_copy(
          bufs[b].at[:, pl.ds(0, HIDDEN)],
          out_hbm.at[pl.ds(base + c * CHUNK, CHUNK)],
          w_sems[b],
      ).wait()

    for b in range(NBUF):
      gather(b, b)

    @pl.loop(NBUF, NSTEPS, step=NBUF)
    def _(c):
      for b in range(NBUF):
        gather_wait(c - NBUF + b, b)
        wb(c - NBUF + b, b)
      for b in range(NBUF):
        wb_wait(c - NBUF + b, b)
        gather(c + b, b)

    for b in range(NBUF):
      gather_wait(NSTEPS - NBUF + b, b)
      wb(NSTEPS - NBUF + b, b)
    for b in range(NBUF):
      wb_wait(NSTEPS - NBUF + b, b)

  return k


RBLK = 512
CBLK = 640


def _tc_out_transpose():
  def body(in_ref, out_ref):
    out_ref[...] = in_ref[...].T

  rows, cols = BATCH, HIST * HIDDEN
  return pl.pallas_call(
      body,
      grid=(rows // RBLK, cols // CBLK),
      in_specs=[pl.BlockSpec((RBLK, CBLK), lambda i, j: (i, j))],
      out_specs=pl.BlockSpec((CBLK, RBLK), lambda i, j: (j, i)),
      out_shape=jax.ShapeDtypeStruct((cols, rows), jnp.float32),
  )


_table_transpose_call = _tc_table_transpose()
_gather_call = _sc_gather()
_out_transpose_call = _tc_out_transpose()


@jax.jit
def kernel(v, weight):
  wt = jnp.swapaxes(weight, 0, 1)
  table = _table_transpose_call(wt)
  idx = v.reshape(NW, NSTEPS, CHUNK)
  rows = _gather_call(idx, table)
  flat = rows.reshape(BATCH, HIST * HIDDEN)
  out_t = _out_transpose_call(flat)
  return out_t.reshape(HIST, HIDDEN, BATCH).transpose(2, 0, 1)


# speedup vs baseline: 2.2741x; 2.2741x over previous
import functools

import jax
import jax.numpy as jnp
from jax import lax
from jax.experimental import pallas as pl
from jax.experimental.pallas import tpu as pltpu
from jax.experimental.pallas import tpu_sc as plsc

VOCAB = 1000000
HIDDEN = 64
BATCH = 16384
HIST = 50

NC = 2
NS = 16
NW = NC * NS

CHUNK = 128
IPW = BATCH // NW
TPW = IPW // CHUNK
NBUF = 4

PAD = 128
TBLK = 8192


def _tc_table_transpose():
  def body(wt_ref, out_ref):
    out_ref[:, 0:HIDDEN] = wt_ref[...].T

  grid = (VOCAB + TBLK - 1) // TBLK
  return pl.pallas_call(
      body,
      grid=(grid,),
      in_specs=[pl.BlockSpec((HIDDEN, TBLK), lambda i: (0, i))],
      out_specs=pl.BlockSpec((TBLK, PAD), lambda i: (i, 0)),
      out_shape=jax.ShapeDtypeStruct((VOCAB, PAD), jnp.float32),
  )


def _sc_gather():
  mesh = plsc.VectorSubcoreMesh(
      core_axis_name="c", subcore_axis_name="s", num_cores=NC, num_subcores=NS
  )

  @functools.partial(
      pl.kernel,
      out_type=jax.ShapeDtypeStruct((HIST, BATCH, PAD), jnp.float32),
      mesh=mesh,
      scratch_types=[
          pltpu.VMEM((HIST, TPW, CHUNK), jnp.int32),
          [pltpu.VMEM((CHUNK, PAD), jnp.float32) for _ in range(NBUF)],
          [pltpu.SemaphoreType.DMA for _ in range(NBUF)],
          [pltpu.SemaphoreType.DMA for _ in range(NBUF)],
      ],
      compiler_params=pltpu.CompilerParams(use_tc_tiling_on_sc=True),
  )
  def k(idx_hbm, table_hbm, out_hbm, idx_v, bufs, g_sems, w_sems):
    wid = lax.axis_index("s") * NC + lax.axis_index("c")
    ibase = wid * IPW
    pltpu.sync_copy(idx_hbm.at[:, pl.ds(wid * TPW, TPW)], idx_v)

    def gather(c, b):
      pltpu.async_copy(
          table_hbm.at[idx_v.at[c // TPW, c % TPW]], bufs[b], g_sems[b]
      )

    def gather_wait(c, b):
      pltpu.make_async_copy(
          table_hbm.at[idx_v.at[c // TPW, c % TPW]], bufs[b], g_sems[b]
      ).wait()

    def _dst(c):
      return out_hbm.at[c // TPW, pl.ds(ibase + (c % TPW) * CHUNK, CHUNK), :]

    def wb(c, b):
      pltpu.async_copy(bufs[b], _dst(c), w_sems[b])

    def wb_wait(c, b):
      pltpu.make_async_copy(bufs[b], _dst(c), w_sems[b]).wait()

    nsteps = HIST * TPW
    for b in range(NBUF):
      gather(b, b)

    @pl.loop(NBUF, nsteps, step=NBUF)
    def _(c):
      for b in range(NBUF):
        gather_wait(c - NBUF + b, b)
        wb(c - NBUF + b, b)
      for b in range(NBUF):
        wb_wait(c - NBUF + b, b)
        gather(c + b, b)

    for b in range(NBUF):
      gather_wait(nsteps - NBUF + b, b)
      wb(nsteps - NBUF + b, b)
    for b in range(NBUF):
      wb_wait(nsteps - NBUF + b, b)

  return k


IB = 4096


def _tc_out_transpose():
  def body(in_ref, out_ref):
    out_ref[...] = in_ref[0, :, 0:HIDDEN].T

  return pl.pallas_call(
      body,
      grid=(HIST, BATCH // IB),
      in_specs=[pl.BlockSpec((1, IB, PAD), lambda j, b: (j, b, 0))],
      out_specs=pl.BlockSpec((HIDDEN, IB), lambda j, b: (j, b)),
      out_shape=jax.ShapeDtypeStruct((HIST * HIDDEN, BATCH), jnp.float32),
  )


_table_transpose_call = _tc_table_transpose()
_gather_call = _sc_gather()
_out_transpose_call = _tc_out_transpose()


@jax.jit
def kernel(v, weight):
  wt = jnp.swapaxes(weight, 0, 1)
  table = _table_transpose_call(wt)
  idx = jnp.swapaxes(v, 0, 1).reshape(HIST, BATCH // CHUNK, CHUNK)
  rows = _gather_call(idx, table)
  out_t = _out_transpose_call(rows)
  return out_t.reshape(HIST, HIDDEN, BATCH).transpose(2, 0, 1)
